# baseline (speedup 1.0000x reference)
import jax
import jax.numpy as jnp
from jax.experimental import pallas as pl
from jax.experimental.pallas import tpu as pltpu

HIDDEN_DIM = 768
SEQ_PAD = 80
BATCH = 1024
BB = 32


def _body(o_ref):
    o_ref[...] = jnp.full((BB, SEQ_PAD, HIDDEN_DIM), 0.5, jnp.float32)


def kernel(species, W, gamma, beta):
    out = pl.pallas_call(
        _body,
        grid=(BATCH // BB,),
        out_specs=pl.BlockSpec((BB, SEQ_PAD, HIDDEN_DIM), lambda i: (i, 0, 0)),
        out_shape=jax.ShapeDtypeStruct((BATCH, SEQ_PAD, HIDDEN_DIM), jnp.float32),
        compiler_params=pltpu.CompilerParams(
            dimension_semantics=("arbitrary",),
        ),
    )()
    return out[:, :77, :]


# speedup vs baseline: 1.6687x; 1.6687x over previous
import jax
import jax.numpy as jnp
from jax.experimental import pallas as pl
from jax.experimental.pallas import tpu as pltpu

NUM_CLASSES = 1000
HIDDEN_DIM = 768
SEQ_LEN = 77
BATCH = 1024
BB = 8


def _body(species_ref, *refs):
    row_refs = refs[:BB]
    g_ref, b_ref, o_ref = refs[BB], refs[BB + 1], refs[BB + 2]
    rows = jnp.concatenate([r[0] for r in row_refs], axis=0)
    mu = jnp.mean(rows, axis=-1, keepdims=True)
    var = jnp.mean(jnp.square(rows - mu), axis=-1, keepdims=True)
    o_ref[...] = (rows - mu) * jax.lax.rsqrt(var + 1e-5) * g_ref[...] + b_ref[...]


def kernel(species, W, gamma, beta):
    species = species.astype(jnp.int32)
    gamma2 = gamma.reshape(1, HIDDEN_DIM)
    beta2 = beta.reshape(1, HIDDEN_DIM)

    W3 = W.reshape(NUM_CLASSES, 1, HIDDEN_DIM)

    def row_spec(r):
        return pl.BlockSpec(
            (1, 1, HIDDEN_DIM), lambda i, s, r=r: (s[i * BB + r], 0, 0)
        )

    grid_spec = pltpu.PrefetchScalarGridSpec(
        num_scalar_prefetch=1,
        grid=(BATCH // BB,),
        in_specs=[row_spec(r) for r in range(BB)]
        + [
            pl.BlockSpec((1, HIDDEN_DIM), lambda i, s: (0, 0)),
            pl.BlockSpec((1, HIDDEN_DIM), lambda i, s: (0, 0)),
        ],
        out_specs=pl.BlockSpec((BB, HIDDEN_DIM), lambda i, s: (i, 0)),
    )
    emb = pl.pallas_call(
        _body,
        grid_spec=grid_spec,
        out_shape=jax.ShapeDtypeStruct((BATCH, HIDDEN_DIM), jnp.float32),
        compiler_params=pltpu.CompilerParams(
            dimension_semantics=("arbitrary",),
        ),
    )(species, *([W3] * BB), gamma2, beta2)
    return jnp.broadcast_to(emb[:, None, :], (BATCH, SEQ_LEN, HIDDEN_DIM))


# speedup vs baseline: 2.2826x; 1.3679x over previous
import jax
import jax.numpy as jnp
from jax.experimental import pallas as pl
from jax.experimental.pallas import tpu as pltpu

NUM_CLASSES = 1000
HIDDEN_DIM = 768
SEQ_LEN = 77
BATCH = 1024
BB = 16


def _body(species_ref, w_ref, g_ref, b_ref, o_ref):
    i = pl.program_id(0)
    rows = jnp.concatenate(
        [w_ref[pl.ds(species_ref[i * BB + r], 1), :] for r in range(BB)], axis=0
    )
    mu = jnp.mean(rows, axis=-1, keepdims=True)
    var = jnp.mean(jnp.square(rows - mu), axis=-1, keepdims=True)
    o_ref[...] = (rows - mu) * jax.lax.rsqrt(var + 1e-5) * g_ref[...] + b_ref[...]


def kernel(species, W, gamma, beta):
    species = species.astype(jnp.int32)
    grid_spec = pltpu.PrefetchScalarGridSpec(
        num_scalar_prefetch=1,
        grid=(BATCH // BB,),
        in_specs=[
            pl.BlockSpec((NUM_CLASSES, HIDDEN_DIM), lambda i, s: (0, 0)),
            pl.BlockSpec((1, HIDDEN_DIM), lambda i, s: (0, 0)),
            pl.BlockSpec((1, HIDDEN_DIM), lambda i, s: (0, 0)),
        ],
        out_specs=pl.BlockSpec((BB, HIDDEN_DIM), lambda i, s: (i, 0)),
    )
    emb = pl.pallas_call(
        _body,
        grid_spec=grid_spec,
        out_shape=jax.ShapeDtypeStruct((BATCH, HIDDEN_DIM), jnp.float32),
        compiler_params=pltpu.CompilerParams(
            dimension_semantics=("arbitrary",),
        ),
    )(species, W, gamma.reshape(1, HIDDEN_DIM), beta.reshape(1, HIDDEN_DIM))
    return jax.lax.broadcast_in_dim(emb, (BATCH, SEQ_LEN, HIDDEN_DIM), (0, 2))


# speedup vs baseline: 2.5261x; 1.1067x over previous
import jax
import jax.numpy as jnp
from jax.experimental import pallas as pl
from jax.experimental.pallas import tpu as pltpu

NUM_CLASSES = 1000
HIDDEN_DIM = 768
SEQ_LEN = 77
BATCH = 1024
BB = 16


def _body(species_ref, w_ref, g_ref, b_ref, o_ref):
    i = pl.program_id(0)
    rows = jnp.concatenate(
        [w_ref[pl.ds(species_ref[i * BB + r], 1), :] for r in range(BB)], axis=0
    )
    mu = jnp.mean(rows, axis=-1, keepdims=True)
    var = jnp.mean(jnp.square(rows - mu), axis=-1, keepdims=True)
    o_ref[pl.ds(i * BB, BB), :] = (
        (rows - mu) * jax.lax.rsqrt(var + 1e-5) * g_ref[...] + b_ref[...]
    )


def kernel(species, W, gamma, beta):
    species = species.astype(jnp.int32)
    grid_spec = pltpu.PrefetchScalarGridSpec(
        num_scalar_prefetch=1,
        grid=(BATCH // BB,),
        in_specs=[
            pl.BlockSpec((NUM_CLASSES, HIDDEN_DIM), lambda i, s: (0, 0)),
            pl.BlockSpec((1, HIDDEN_DIM), lambda i, s: (0, 0)),
            pl.BlockSpec((1, HIDDEN_DIM), lambda i, s: (0, 0)),
        ],
        out_specs=pl.BlockSpec(memory_space=pltpu.MemorySpace.VMEM),
    )
    emb = pl.pallas_call(
        _body,
        grid_spec=grid_spec,
        out_shape=jax.ShapeDtypeStruct((BATCH, HIDDEN_DIM), jnp.float32),
        compiler_params=pltpu.CompilerParams(
            dimension_semantics=("arbitrary",),
        ),
    )(species, W, gamma.reshape(1, HIDDEN_DIM), beta.reshape(1, HIDDEN_DIM))
    return jax.lax.broadcast_in_dim(emb, (BATCH, SEQ_LEN, HIDDEN_DIM), (0, 2))


# speedup vs baseline: 2.7763x; 1.0990x over previous
import jax
import jax.numpy as jnp
from jax.experimental import pallas as pl
from jax.experimental.pallas import tpu as pltpu

NUM_CLASSES = 1000
HIDDEN_DIM = 768
SEQ_LEN = 77
BATCH = 1024
BB = 32


def _body(species_ref, w_ref, g_ref, b_ref, o_ref):
    i = pl.program_id(0)
    rows = jnp.concatenate(
        [w_ref[pl.ds(species_ref[i * BB + r], 1), :] for r in range(BB)], axis=0
    )
    mu = jnp.mean(rows, axis=-1, keepdims=True)
    var = jnp.mean(jnp.square(rows - mu), axis=-1, keepdims=True)
    o_ref[pl.ds(i * BB, BB), :] = (
        (rows - mu) * jax.lax.rsqrt(var + 1e-5) * g_ref[...] + b_ref[...]
    )


def kernel(species, W, gamma, beta):
    species = species.astype(jnp.int32)
    grid_spec = pltpu.PrefetchScalarGridSpec(
        num_scalar_prefetch=1,
        grid=(BATCH // BB,),
        in_specs=[
            pl.BlockSpec((NUM_CLASSES, HIDDEN_DIM), lambda i, s: (0, 0)),
            pl.BlockSpec((1, HIDDEN_DIM), lambda i, s: (0, 0)),
            pl.BlockSpec((1, HIDDEN_DIM), lambda i, s: (0, 0)),
        ],
        out_specs=pl.BlockSpec(memory_space=pltpu.MemorySpace.VMEM),
    )
    emb = pl.pallas_call(
        _body,
        grid_spec=grid_spec,
        out_shape=jax.ShapeDtypeStruct((BATCH, HIDDEN_DIM), jnp.float32),
        compiler_params=pltpu.CompilerParams(
            dimension_semantics=("arbitrary",),
        ),
    )(species, W, gamma.reshape(1, HIDDEN_DIM), beta.reshape(1, HIDDEN_DIM))
    return jax.lax.broadcast_in_dim(emb, (BATCH, SEQ_LEN, HIDDEN_DIM), (0, 2))
